# baseline (speedup 1.0000x reference)
import jax
import jax.numpy as jnp
from jax import lax
from jax.experimental import pallas as pl
from jax.experimental.pallas import tpu as pltpu, tpu_sc as plsc

B, S = 128, 512
V, H, P, T = 30522, 768, 512, 2
EPS = 1e-12

NW = 32
C = 32
NCHUNK_S = S // C
BG = B // (NW // NCHUNK_S)
NV = H // 16
UNROLL = 4


def _body(ids_hbm, tt_hbm, word_hbm, pos_hbm, typ_hbm, gamma_hbm, beta_hbm,
          out_hbm, idx_all, tt_all, pos_v, rows_a, rows_b, typ_v, tdiff_v,
          gamma_v, beta_v, gsem_a, gsem_b):
    wid = lax.axis_index("s") * 2 + lax.axis_index("c")
    c = lax.rem(wid, NCHUNK_S)
    bg = lax.div(wid, NCHUNK_S)
    s0 = c * C
    b0 = bg * BG

    pltpu.sync_copy(pos_hbm.at[pl.ds(s0, C)], pos_v)
    pltpu.sync_copy(typ_hbm, typ_v)
    pltpu.sync_copy(gamma_hbm, gamma_v)
    pltpu.sync_copy(beta_hbm, beta_v)
    pltpu.sync_copy(ids_hbm.at[c, pl.ds(b0 * C, BG * C)], idx_all)
    pltpu.sync_copy(tt_hbm.at[c, pl.ds(b0 * C, BG * C)], tt_all)

    for j in range(NV):
        sl = pl.ds(j * 16, 16)
        tdiff_v[sl] = typ_v[1, sl] - typ_v[0, sl]

    def prep(t, carry):
        for j in range(NV):
            sl = pl.ds(j * 16, 16)
            pos_v[t, sl] = pos_v[t, sl] + typ_v[0, sl]
        return carry

    lax.fori_loop(0, C, prep, 0)

    def gstart(i, buf, sem):
        pltpu.async_copy(word_hbm.at[idx_all.at[pl.ds(i * C, C)]], buf, sem)

    def gwait(i, buf, sem):
        pltpu.make_async_copy(
            word_hbm.at[idx_all.at[pl.ds(i * C, C)]], buf, sem).wait()

    def norm_token(i, t, buf):
        ttf = plsc.load_gather(
            tt_all, [jnp.full((16,), i * C + t, jnp.int32)]).astype(
                jnp.float32)
        acc = jnp.zeros((16,), jnp.float32)
        acc2 = jnp.zeros((16,), jnp.float32)
        for j in range(NV):
            sl = pl.ds(j * 16, 16)
            v = buf[t, sl] + (pos_v[t, sl] + ttf * tdiff_v[sl])
            buf[t, sl] = v
            acc = acc + v
            acc2 = acc2 + v * v
        mean = jnp.sum(acc) * (1.0 / H)
        var = jnp.sum(acc2) * (1.0 / H) - mean * mean + EPS
        seed_i = jnp.int32(0x5F3759DF) - lax.shift_right_logical(
            lax.bitcast_convert_type(var, jnp.int32), 1)
        y = lax.bitcast_convert_type(seed_i, jnp.float32)
        y = y * (1.5 - 0.5 * var * y * y)
        y = y * (1.5 - 0.5 * var * y * y)
        y = y * (1.5 - 0.5 * var * y * y)
        for j in range(NV):
            sl = pl.ds(j * 16, 16)
            nv = (buf[t, sl] - mean) * y
            buf[t, sl] = nv * gamma_v[sl] + beta_v[sl]

    def process(i, buf, sem):
        gwait(i, buf, sem)

        def tok_body(tq, carry):
            for u in range(UNROLL):
                norm_token(i, tq * UNROLL + u, buf)
            return carry

        lax.fori_loop(0, C // UNROLL, tok_body, 0)
        pltpu.sync_copy(buf, out_hbm.at[pl.ds((b0 + i) * S + s0, C)])

        @pl.when(i < BG - 2)
        def _():
            gstart(i + 2, buf, sem)

    gstart(0, rows_a, gsem_a)
    gstart(1, rows_b, gsem_b)

    def pair_body(p, carry):
        process(2 * p, rows_a, gsem_a)
        process(2 * p + 1, rows_b, gsem_b)
        return carry

    lax.fori_loop(0, BG // 2, pair_body, 0)


@jax.jit
def _run(input_ids, token_type_ids, word_emb, pos_emb, type_emb, gamma, beta):
    mesh = plsc.VectorSubcoreMesh(core_axis_name="c", subcore_axis_name="s")
    f = pl.kernel(
        _body,
        out_type=jax.ShapeDtypeStruct((B * S, H), jnp.float32),
        mesh=mesh,
        scratch_types=[
            pltpu.VMEM((BG * C,), jnp.int32),
            pltpu.VMEM((BG * C,), jnp.int32),
            pltpu.VMEM((C, H), jnp.float32),
            pltpu.VMEM((C, H), jnp.float32),
            pltpu.VMEM((C, H), jnp.float32),
            pltpu.VMEM((T, H), jnp.float32),
            pltpu.VMEM((H,), jnp.float32),
            pltpu.VMEM((H,), jnp.float32),
            pltpu.VMEM((H,), jnp.float32),
            pltpu.SemaphoreType.DMA,
            pltpu.SemaphoreType.DMA,
        ],
        compiler_params=pltpu.CompilerParams(needs_layout_passes=False),
    )
    return f(input_ids, token_type_ids, word_emb, pos_emb, type_emb, gamma,
             beta)


def kernel(input_ids, token_type_ids, word_emb, pos_emb, type_emb, gamma,
           beta):
    ids = input_ids.astype(jnp.int32).reshape(B, NCHUNK_S, C) \
        .swapaxes(0, 1).reshape(NCHUNK_S, B * C)
    tt = token_type_ids.astype(jnp.int32).reshape(B, NCHUNK_S, C) \
        .swapaxes(0, 1).reshape(NCHUNK_S, B * C)
    out = _run(ids, tt, word_emb, pos_emb, type_emb, gamma, beta)
    return out.reshape(B, S, H)


# speedup vs baseline: 2.2058x; 2.2058x over previous
import jax
import jax.numpy as jnp
from jax import lax
from jax.experimental import pallas as pl
from jax.experimental.pallas import tpu as pltpu, tpu_sc as plsc

B, S = 128, 512
V, H, P, T = 30522, 768, 512, 2
EPS = 1e-12

NW = 32
C = 32
NCHUNK_S = S // C
BG = B // (NW // NCHUNK_S)
NV = H // 16
UNROLL = 2


def _body(ids_hbm, tt_hbm, word_hbm, pos_hbm, typ_hbm, gamma_hbm, beta_hbm,
          out_hbm, idx_all, tt_all, pos_v, rows_a, rows_b, typ_v, tdiff_v,
          gamma_v, beta_v, gsem_a, gsem_b):
    wid = lax.axis_index("s") * 2 + lax.axis_index("c")
    c = lax.rem(wid, NCHUNK_S)
    bg = lax.div(wid, NCHUNK_S)
    s0 = c * C
    b0 = bg * BG

    pltpu.sync_copy(pos_hbm.at[pl.ds(s0, C)], pos_v)
    pltpu.sync_copy(typ_hbm, typ_v)
    pltpu.sync_copy(gamma_hbm, gamma_v)
    pltpu.sync_copy(beta_hbm, beta_v)
    pltpu.sync_copy(ids_hbm.at[c, pl.ds(b0 * C, BG * C)], idx_all)
    pltpu.sync_copy(tt_hbm.at[c, pl.ds(b0 * C, BG * C)], tt_all)

    for j in range(NV):
        sl = pl.ds(j * 16, 16)
        tdiff_v[sl] = typ_v[1, sl] - typ_v[0, sl]

    def prep(t, carry):
        for j in range(NV):
            sl = pl.ds(j * 16, 16)
            pos_v[t, sl] = pos_v[t, sl] + typ_v[0, sl]
        return carry

    lax.fori_loop(0, C, prep, 0)

    def gstart(i, buf, sem):
        pltpu.async_copy(word_hbm.at[idx_all.at[pl.ds(i * C, C)]], buf, sem)

    def gwait(i, buf, sem):
        pltpu.make_async_copy(
            word_hbm.at[idx_all.at[pl.ds(i * C, C)]], buf, sem).wait()

    def norm_token(i, t, buf):
        ttf = plsc.load_gather(
            tt_all, [jnp.full((16,), i * C + t, jnp.int32)]).astype(
                jnp.float32)
        acc = jnp.zeros((16,), jnp.float32)
        acc2 = jnp.zeros((16,), jnp.float32)
        for j in range(NV):
            sl = pl.ds(j * 16, 16)
            v = buf[t, sl] + (pos_v[t, sl] + ttf * tdiff_v[sl])
            buf[t, sl] = v
            acc = acc + v
            acc2 = acc2 + v * v
        mean = jnp.sum(acc) * (1.0 / H)
        var = jnp.sum(acc2) * (1.0 / H) - mean * mean + EPS
        seed_i = jnp.int32(0x5F3759DF) - lax.shift_right_logical(
            lax.bitcast_convert_type(var, jnp.int32), 1)
        y = lax.bitcast_convert_type(seed_i, jnp.float32)
        y = y * (1.5 - 0.5 * var * y * y)
        y = y * (1.5 - 0.5 * var * y * y)
        y = y * (1.5 - 0.5 * var * y * y)
        for j in range(NV):
            sl = pl.ds(j * 16, 16)
            nv = (buf[t, sl] - mean) * y
            buf[t, sl] = nv * gamma_v[sl] + beta_v[sl]

    def process(i, buf, sem):
        gwait(i, buf, sem)

        @plsc.parallel_loop(0, C, step=1, unroll=UNROLL)
        def tok_body(t):
            norm_token(i, t, buf)
        pltpu.sync_copy(buf, out_hbm.at[pl.ds((b0 + i) * S + s0, C)])

        @pl.when(i < BG - 2)
        def _():
            gstart(i + 2, buf, sem)

    gstart(0, rows_a, gsem_a)
    gstart(1, rows_b, gsem_b)

    def pair_body(p, carry):
        process(2 * p, rows_a, gsem_a)
        process(2 * p + 1, rows_b, gsem_b)
        return carry

    lax.fori_loop(0, BG // 2, pair_body, 0)


@jax.jit
def _run(input_ids, token_type_ids, word_emb, pos_emb, type_emb, gamma, beta):
    mesh = plsc.VectorSubcoreMesh(core_axis_name="c", subcore_axis_name="s")
    f = pl.kernel(
        _body,
        out_type=jax.ShapeDtypeStruct((B * S, H), jnp.float32),
        mesh=mesh,
        scratch_types=[
            pltpu.VMEM((BG * C,), jnp.int32),
            pltpu.VMEM((BG * C,), jnp.int32),
            pltpu.VMEM((C, H), jnp.float32),
            pltpu.VMEM((C, H), jnp.float32),
            pltpu.VMEM((C, H), jnp.float32),
            pltpu.VMEM((T, H), jnp.float32),
            pltpu.VMEM((H,), jnp.float32),
            pltpu.VMEM((H,), jnp.float32),
            pltpu.VMEM((H,), jnp.float32),
            pltpu.SemaphoreType.DMA,
            pltpu.SemaphoreType.DMA,
        ],
        compiler_params=pltpu.CompilerParams(needs_layout_passes=False),
    )
    return f(input_ids, token_type_ids, word_emb, pos_emb, type_emb, gamma,
             beta)


def kernel(input_ids, token_type_ids, word_emb, pos_emb, type_emb, gamma,
           beta):
    ids = input_ids.astype(jnp.int32).reshape(B, NCHUNK_S, C) \
        .swapaxes(0, 1).reshape(NCHUNK_S, B * C)
    tt = token_type_ids.astype(jnp.int32).reshape(B, NCHUNK_S, C) \
        .swapaxes(0, 1).reshape(NCHUNK_S, B * C)
    out = _run(ids, tt, word_emb, pos_emb, type_emb, gamma, beta)
    return out.reshape(B, S, H)


# speedup vs baseline: 2.3907x; 1.0838x over previous
import jax
import jax.numpy as jnp
from jax import lax
from jax.experimental import pallas as pl
from jax.experimental.pallas import tpu as pltpu, tpu_sc as plsc

B, S = 128, 512
V, H, P, T = 30522, 768, 512, 2
EPS = 1e-12

NW = 32
C = 32
NCHUNK_S = S // C
BG = B // (NW // NCHUNK_S)
NV = H // 16
UNROLL = 2


def _body(ids_hbm, tt_hbm, word_hbm, pos_hbm, typ_hbm, gamma_hbm, beta_hbm,
          out_hbm, idx_all, tt_all, pos_v, rows_a, rows_b, vbuf_v, typ_v,
          tdiff_v, gamma_v, beta_v, gsem_a, gsem_b):
    wid = lax.axis_index("s") * 2 + lax.axis_index("c")
    c = lax.rem(wid, NCHUNK_S)
    bg = lax.div(wid, NCHUNK_S)
    s0 = c * C
    b0 = bg * BG

    pltpu.sync_copy(pos_hbm.at[pl.ds(s0, C)], pos_v)
    pltpu.sync_copy(typ_hbm, typ_v)
    pltpu.sync_copy(gamma_hbm, gamma_v)
    pltpu.sync_copy(beta_hbm, beta_v)
    pltpu.sync_copy(ids_hbm.at[c, pl.ds(b0 * C, BG * C)], idx_all)
    pltpu.sync_copy(tt_hbm.at[c, pl.ds(b0 * C, BG * C)], tt_all)

    for j in range(NV):
        sl = pl.ds(j * 16, 16)
        tdiff_v[sl] = typ_v[1, sl] - typ_v[0, sl]

    def prep(t, carry):
        for j in range(NV):
            sl = pl.ds(j * 16, 16)
            pos_v[t, sl] = pos_v[t, sl] + typ_v[0, sl]
        return carry

    lax.fori_loop(0, C, prep, 0)

    def gstart(i, buf, sem):
        pltpu.async_copy(word_hbm.at[idx_all.at[pl.ds(i * C, C)]], buf, sem)

    def gwait(i, buf, sem):
        pltpu.make_async_copy(
            word_hbm.at[idx_all.at[pl.ds(i * C, C)]], buf, sem).wait()

    def norm_token(i, t, buf):
        ttf = plsc.load_gather(
            tt_all, [jnp.full((16,), i * C + t, jnp.int32)]).astype(
                jnp.float32)
        acc0 = jnp.zeros((16,), jnp.float32)
        acc1 = jnp.zeros((16,), jnp.float32)
        sq0 = jnp.zeros((16,), jnp.float32)
        sq1 = jnp.zeros((16,), jnp.float32)
        for j in range(NV):
            sl = pl.ds(j * 16, 16)
            v = buf[t, sl] + (pos_v[t, sl] + ttf * tdiff_v[sl])
            vbuf_v[t, sl] = v
            if j % 2 == 0:
                acc0 = acc0 + v
                sq0 = sq0 + v * v
            else:
                acc1 = acc1 + v
                sq1 = sq1 + v * v
        mean = jnp.sum(acc0 + acc1) * (1.0 / H)
        var = jnp.sum(sq0 + sq1) * (1.0 / H) - mean * mean + EPS
        seed_i = jnp.int32(0x5F3759DF) - lax.shift_right_logical(
            lax.bitcast_convert_type(var, jnp.int32), 1)
        y = lax.bitcast_convert_type(seed_i, jnp.float32)
        y = y * (1.5 - 0.5 * var * y * y)
        y = y * (1.5 - 0.5 * var * y * y)
        y = y * (1.5 - 0.5 * var * y * y)
        shift = mean * y
        for j in range(NV):
            sl = pl.ds(j * 16, 16)
            nv = vbuf_v[t, sl] * y - shift
            buf[t, sl] = nv * gamma_v[sl] + beta_v[sl]

    def process(i, buf, sem):
        gwait(i, buf, sem)

        @plsc.parallel_loop(0, C, step=1, unroll=UNROLL)
        def tok_body(t):
            norm_token(i, t, buf)
        pltpu.sync_copy(buf, out_hbm.at[pl.ds((b0 + i) * S + s0, C)])

        @pl.when(i < BG - 2)
        def _():
            gstart(i + 2, buf, sem)

    gstart(0, rows_a, gsem_a)
    gstart(1, rows_b, gsem_b)

    def pair_body(p, carry):
        process(2 * p, rows_a, gsem_a)
        process(2 * p + 1, rows_b, gsem_b)
        return carry

    lax.fori_loop(0, BG // 2, pair_body, 0)


@jax.jit
def _run(input_ids, token_type_ids, word_emb, pos_emb, type_emb, gamma, beta):
    mesh = plsc.VectorSubcoreMesh(core_axis_name="c", subcore_axis_name="s")
    f = pl.kernel(
        _body,
        out_type=jax.ShapeDtypeStruct((B * S, H), jnp.float32),
        mesh=mesh,
        scratch_types=[
            pltpu.VMEM((BG * C,), jnp.int32),
            pltpu.VMEM((BG * C,), jnp.int32),
            pltpu.VMEM((C, H), jnp.float32),
            pltpu.VMEM((C, H), jnp.float32),
            pltpu.VMEM((C, H), jnp.float32),
            pltpu.VMEM((C, H), jnp.float32),
            pltpu.VMEM((T, H), jnp.float32),
            pltpu.VMEM((H,), jnp.float32),
            pltpu.VMEM((H,), jnp.float32),
            pltpu.VMEM((H,), jnp.float32),
            pltpu.SemaphoreType.DMA,
            pltpu.SemaphoreType.DMA,
        ],
        compiler_params=pltpu.CompilerParams(needs_layout_passes=False),
    )
    return f(input_ids, token_type_ids, word_emb, pos_emb, type_emb, gamma,
             beta)


def kernel(input_ids, token_type_ids, word_emb, pos_emb, type_emb, gamma,
           beta):
    ids = input_ids.astype(jnp.int32).reshape(B, NCHUNK_S, C) \
        .swapaxes(0, 1).reshape(NCHUNK_S, B * C)
    tt = token_type_ids.astype(jnp.int32).reshape(B, NCHUNK_S, C) \
        .swapaxes(0, 1).reshape(NCHUNK_S, B * C)
    out = _run(ids, tt, word_emb, pos_emb, type_emb, gamma, beta)
    return out.reshape(B, S, H)


# speedup vs baseline: 10.6136x; 4.4395x over previous
import functools

import jax
import jax.numpy as jnp
from jax import lax
from jax.experimental import pallas as pl
from jax.experimental.pallas import tpu as pltpu, tpu_sc as plsc

B, S = 128, 512
V, H, P, T = 30522, 768, 512, 2
EPS = 1e-12

NW = 32
TOK = B * S
TPW = TOK // NW
C = 64
NC = TPW // C


def _gather_body(ids_hbm, word_hbm, out_hbm, idx_all, buf_a, buf_b,
                 gsem_a, gsem_b, osem_a, osem_b):
    wid = lax.axis_index("s") * 2 + lax.axis_index("c")
    tok0 = wid * TPW
    pltpu.sync_copy(ids_hbm.at[pl.ds(tok0, TPW)], idx_all)

    def gstart(i, buf, sem):
        pltpu.async_copy(word_hbm.at[idx_all.at[pl.ds(i * C, C)]], buf, sem)

    def gwait(i, buf, sem):
        pltpu.make_async_copy(
            word_hbm.at[idx_all.at[pl.ds(i * C, C)]], buf, sem).wait()

    def ostart(i, buf, sem):
        return pltpu.async_copy(
            buf, out_hbm.at[pl.ds(tok0 + i * C, C)], sem)

    gstart(0, buf_a, gsem_a)
    gstart(1, buf_b, gsem_b)

    def pair_body(p, carry):
        i = 2 * p
        gwait(i, buf_a, gsem_a)
        oa = ostart(i, buf_a, osem_a)
        gwait(i + 1, buf_b, gsem_b)
        ob = ostart(i + 1, buf_b, osem_b)
        oa.wait()

        @pl.when(i + 2 < NC)
        def _():
            gstart(i + 2, buf_a, gsem_a)

        ob.wait()

        @pl.when(i + 3 < NC)
        def _():
            gstart(i + 3, buf_b, gsem_b)

        return carry

    lax.fori_loop(0, NC // 2, pair_body, 0)


def _sc_gather(ids_flat, word_emb):
    mesh = plsc.VectorSubcoreMesh(core_axis_name="c", subcore_axis_name="s")
    f = pl.kernel(
        _gather_body,
        out_type=jax.ShapeDtypeStruct((TOK, H), jnp.float32),
        mesh=mesh,
        scratch_types=[
            pltpu.VMEM((TPW,), jnp.int32),
            pltpu.VMEM((C, H), jnp.float32),
            pltpu.VMEM((C, H), jnp.float32),
            pltpu.SemaphoreType.DMA,
            pltpu.SemaphoreType.DMA,
            pltpu.SemaphoreType.DMA,
            pltpu.SemaphoreType.DMA,
        ],
        compiler_params=pltpu.CompilerParams(needs_layout_passes=False),
    )
    return f(ids_flat, word_emb)


def _ln_kernel(tt_ref, x_ref, pos_ref, typ_ref, gamma_ref, beta_ref, o_ref):
    x = x_ref[...]
    ttf = tt_ref[0, 0, :]
    t0 = typ_ref[0, :]
    tdiff = typ_ref[1, :] - typ_ref[0, :]
    v = x + pos_ref[...] + t0[None, :] + ttf[:, None] * tdiff[None, :]
    mean = jnp.mean(v, axis=-1, keepdims=True)
    var = jnp.mean(v * v, axis=-1, keepdims=True) - mean * mean
    inv = lax.rsqrt(var + EPS)
    o_ref[...] = ((v - mean) * inv) * gamma_ref[0, :][None, :] \
        + beta_ref[0, :][None, :]


def _tc_ln(ttf2d, rows, pos_emb, type_emb, gamma, beta):
    return pl.pallas_call(
        _ln_kernel,
        grid=(B,),
        in_specs=[
            pl.BlockSpec((1, 1, S), lambda i: (i, 0, 0)),
            pl.BlockSpec((S, H), lambda i: (i, 0)),
            pl.BlockSpec((S, H), lambda i: (0, 0)),
            pl.BlockSpec((T, H), lambda i: (0, 0)),
            pl.BlockSpec((1, H), lambda i: (0, 0)),
            pl.BlockSpec((1, H), lambda i: (0, 0)),
        ],
        out_specs=pl.BlockSpec((S, H), lambda i: (i, 0)),
        out_shape=jax.ShapeDtypeStruct((TOK, H), jnp.float32),
    )(ttf2d, rows, pos_emb, type_emb, gamma, beta)


@jax.jit
def _run(ids_flat, ttf2d, word_emb, pos_emb, type_emb, gamma2d, beta2d):
    rows = _sc_gather(ids_flat, word_emb)
    return _tc_ln(ttf2d, rows, pos_emb, type_emb, gamma2d, beta2d)


def kernel(input_ids, token_type_ids, word_emb, pos_emb, type_emb, gamma,
           beta):
    ids_flat = input_ids.astype(jnp.int32).reshape(TOK)
    ttf2d = token_type_ids.astype(jnp.float32).reshape(B, 1, S)
    out = _run(ids_flat, ttf2d, word_emb, pos_emb, type_emb,
               gamma.reshape(1, H), beta.reshape(1, H))
    return out.reshape(B, S, H)
